# Initial kernel scaffold; baseline (speedup 1.0000x reference)
import jax
import jax.numpy as jnp
from jax.experimental import pallas as pl


def kernel(pred_sid, pred_beta, pred_energy_corr_factor, recHitID, recHitEnergy):
    raise NotImplementedError("write your pallas kernel here")


# speedup vs baseline: 83.6718x; 83.6718x over previous
import functools

import jax
import jax.numpy as jnp
from jax import lax
from jax.experimental import pallas as pl
from jax.experimental.pallas import tpu as pltpu
from jax.experimental.pallas import tpu_sc as plsc

N = 1_600_000
NC = 2
NS = 16
NW = NC * NS
CHUNK = N // NW
SUB = 2000
NSUB = CHUNK // SUB
VPB = SUB // 16
SEG_P = 5120
SEG_W = 256
NT2 = SEG_P // SEG_W
NEG = float("-inf")

_mesh = plsc.VectorSubcoreMesh(core_axis_name="c", subcore_axis_name="s")
_cparams = pltpu.CompilerParams(needs_layout_passes=False)


def _wid():
    return lax.axis_index("s") * NC + lax.axis_index("c")


def _shift(x, idx):
    return x.at[idx].get(mode="promise_in_bounds")


def _splat(v):
    return jnp.full((16,), v, jnp.int32)


@functools.partial(
    pl.kernel,
    out_type=jax.ShapeDtypeStruct((NW, 8, SEG_P), jnp.float32),
    mesh=_mesh,
    compiler_params=_cparams,
    scratch_types=[
        pltpu.VMEM((SUB,), jnp.int32),
        pltpu.VMEM((SUB,), jnp.float32),
        pltpu.VMEM((SUB,), jnp.float32),
        pltpu.VMEM((SUB,), jnp.int32),
        pltpu.VMEM((SUB,), jnp.float32),
        pltpu.VMEM((8, SEG_P), jnp.float32),
    ],
)
def _phase1(sid_h, beta_h, corr_h, rid_h, en_h, part_h,
            sid_v, beta_v, corr_v, rid_v, en_v, tb):
    wid = _wid()
    neg16 = jnp.full((16,), NEG, jnp.float32)
    zero16 = jnp.zeros((16,), jnp.float32)
    iota = lax.iota(jnp.int32, 16)
    rBH, rCH, rBT, rCT, rEH, rET = (_splat(q) for q in range(6))

    def init_body(i, _):
        s = pl.ds(i * 16, 16)
        tb[0, s] = neg16
        tb[1, s] = neg16
        tb[2, s] = neg16
        tb[3, s] = neg16
        tb[4, s] = zero16
        tb[5, s] = zero16
        return 0

    lax.fori_loop(0, SEG_P // 16, init_body, 0)

    base0 = wid * CHUNK

    def block_body(b, _):
        off = pl.multiple_of(base0 + b * SUB, 8)
        pltpu.sync_copy(sid_h.at[pl.ds(off, SUB)], sid_v)
        pltpu.sync_copy(beta_h.at[pl.ds(off, SUB)], beta_v)
        pltpu.sync_copy(corr_h.at[pl.ds(off, SUB)], corr_v)
        pltpu.sync_copy(rid_h.at[pl.ds(off, SUB)], rid_v)
        pltpu.sync_copy(en_h.at[pl.ds(off, SUB)], en_v)

        def vreg_body(i, _):
            s = pl.ds(i * 16, 16)
            sv = sid_v[s]
            bv = beta_v[s]
            cv = corr_v[s]
            hit = rid_v[s] == 0
            ev = en_v[s]
            bh = jnp.where(hit, bv, neg16)
            ch = jnp.where(hit, cv, neg16)
            bt = jnp.where(hit, neg16, bv)
            ct = jnp.where(hit, neg16, cv)
            eh = jnp.where(hit, ev, zero16)
            et = jnp.where(hit, zero16, ev)
            for k in (1, 2, 4, 8):
                idx = jnp.minimum(iota + k, 15)
                same = (iota < (16 - k)) & (_shift(sv, idx) == sv)
                eh = eh + jnp.where(same, _shift(eh, idx), zero16)
                et = et + jnp.where(same, _shift(et, idx), zero16)
                cb = jnp.where(same, _shift(bh, idx), neg16)
                cc = jnp.where(same, _shift(ch, idx), neg16)
                take = (cb > bh) | ((cb == bh) & (cc > ch))
                bh = jnp.where(take, cb, bh)
                ch = jnp.where(take, cc, ch)
                cb = jnp.where(same, _shift(bt, idx), neg16)
                cc = jnp.where(same, _shift(ct, idx), neg16)
                take = (cb > bt) | ((cb == bt) & (cc > ct))
                bt = jnp.where(take, cb, bt)
                ct = jnp.where(take, cc, ct)
            pidx = jnp.maximum(iota - 1, 0)
            first = (iota == 0) | (_shift(sv, pidx) != sv)
            plsc.addupdate_scatter(tb, [rEH, sv], eh, mask=first)
            plsc.addupdate_scatter(tb, [rET, sv], et, mask=first)
            ob = plsc.load_gather(tb, [rBH, sv])
            oc = plsc.load_gather(tb, [rCH, sv])
            wm = first & ((bh > ob) | ((bh == ob) & (ch > oc)))
            plsc.store_scatter(tb, [rBH, sv], bh, mask=wm)
            plsc.store_scatter(tb, [rCH, sv], ch, mask=wm)
            ob = plsc.load_gather(tb, [rBT, sv])
            oc = plsc.load_gather(tb, [rCT, sv])
            wm = first & ((bt > ob) | ((bt == ob) & (ct > oc)))
            plsc.store_scatter(tb, [rBT, sv], bt, mask=wm)
            plsc.store_scatter(tb, [rCT, sv], ct, mask=wm)
            return 0

        lax.fori_loop(0, VPB, vreg_body, 0)
        return 0

    lax.fori_loop(0, NSUB, block_body, 0)

    pltpu.sync_copy(tb, part_h.at[wid])


@functools.partial(
    pl.kernel,
    out_type=jax.ShapeDtypeStruct((8, SEG_P), jnp.float32),
    mesh=_mesh,
    compiler_params=_cparams,
    scratch_types=[
        pltpu.VMEM((NW, 8, SEG_W), jnp.float32),
        pltpu.VMEM((8, SEG_W), jnp.float32),
    ],
)
def _phase2(part_h, ftab_h, pbuf, obuf):
    wid = _wid()

    @pl.when(wid < NT2)
    def _():
        off = pl.multiple_of(wid * SEG_W, 128)
        pltpu.sync_copy(part_h.at[:, :, pl.ds(off, SEG_W)], pbuf)
        neg16 = jnp.full((16,), NEG, jnp.float32)
        zero16 = jnp.zeros((16,), jnp.float32)
        for g in range(SEG_W // 16):
            s = pl.ds(g * 16, 16)

            def body(p, acc):
                bh, ch, bt, ct, eh, et = acc
                nb = pbuf[p, 0, s]
                nc_ = pbuf[p, 1, s]
                take = (nb > bh) | ((nb == bh) & (nc_ > ch))
                bh = jnp.where(take, nb, bh)
                ch = jnp.where(take, nc_, ch)
                nb = pbuf[p, 2, s]
                nc_ = pbuf[p, 3, s]
                take = (nb > bt) | ((nb == bt) & (nc_ > ct))
                bt = jnp.where(take, nb, bt)
                ct = jnp.where(take, nc_, ct)
                eh = eh + pbuf[p, 4, s]
                et = et + pbuf[p, 5, s]
                return (bh, ch, bt, ct, eh, et)

            bh, ch, bt, ct, eh, et = lax.fori_loop(
                0, NW, body, (neg16, neg16, neg16, neg16, zero16, zero16))
            ch = jnp.where(bh != neg16, ch, zero16)
            ct = jnp.where(bt != neg16, ct, zero16)
            ech = ch * eh
            ect = ct * et
            obuf[0, s] = et
            obuf[1, s] = ect
            obuf[2, s] = eh
            obuf[3, s] = ech
            obuf[4, s] = jnp.where(et != 0.0, et, eh)
            obuf[5, s] = jnp.where(ect != 0.0, ect, ech)
            obuf[6, s] = jnp.where(eh != 0.0, eh, et)
            obuf[7, s] = jnp.where(ech != 0.0, ech, ect)
        pltpu.sync_copy(obuf, ftab_h.at[:, pl.ds(off, SEG_W)])


@functools.partial(
    pl.kernel,
    out_type=[jax.ShapeDtypeStruct((N,), jnp.float32)] * 8,
    mesh=_mesh,
    compiler_params=_cparams,
    scratch_types=[
        pltpu.VMEM((8, SEG_P), jnp.float32),
        pltpu.VMEM((SUB,), jnp.int32),
    ] + [pltpu.VMEM((SUB,), jnp.float32)] * 8,
)
def _phase3(ftab_h, sid_h, o0, o1, o2, o3, o4, o5, o6, o7, t8, sidb,
            b0, b1, b2, b3, b4, b5, b6, b7):
    wid = _wid()
    pltpu.sync_copy(ftab_h, t8)
    outs = (o0, o1, o2, o3, o4, o5, o6, o7)
    bufs = (b0, b1, b2, b3, b4, b5, b6, b7)
    base0 = wid * CHUNK
    rows = [_splat(q) for q in range(8)]

    def block_body(b, _):
        off = pl.multiple_of(base0 + b * SUB, 8)
        pltpu.sync_copy(sid_h.at[pl.ds(off, SUB)], sidb)

        def vreg_body(i, _):
            s = pl.ds(i * 16, 16)
            sv = sidb[s]
            for q in range(8):
                bufs[q][s] = plsc.load_gather(t8, [rows[q], sv])
            return 0

        lax.fori_loop(0, VPB, vreg_body, 0)
        for q in range(8):
            pltpu.sync_copy(bufs[q], outs[q].at[pl.ds(off, SUB)])
        return 0

    lax.fori_loop(0, NSUB, block_body, 0)


@jax.jit
def _run(sid, beta, corr, rid, en):
    part = _phase1(sid, beta, corr, rid, en)
    ftab = _phase2(part)
    return _phase3(ftab, sid)


def kernel(pred_sid, pred_beta, pred_energy_corr_factor, recHitID, recHitEnergy):
    sid = pred_sid.reshape(N).astype(jnp.int32)
    beta = pred_beta.reshape(N).astype(jnp.float32)
    corr = pred_energy_corr_factor.reshape(N).astype(jnp.float32)
    rid = recHitID.reshape(N).astype(jnp.int32)
    en = recHitEnergy.reshape(N).astype(jnp.float32)
    outs = _run(sid, beta, corr, rid, en)
    return tuple(o.reshape(N, 1) for o in outs)
